# Initial kernel scaffold; baseline (speedup 1.0000x reference)
import jax
import jax.numpy as jnp
from jax.experimental import pallas as pl


def kernel(inputs, roi_label, cls_scores, images, proposals, GT_info, lut, cq):
    raise NotImplementedError("write your pallas kernel here")


# speedup vs baseline: 1.6045x; 1.6045x over previous
import jax
import jax.numpy as jnp
from jax.experimental import pallas as pl
from jax.experimental.pallas import tpu as pltpu

_NUM_FEATURES = 128
_NUM_PIDS = 100000
_NUM_CQ = 5000
_OIM_SCALAR = 30.0
_B = 128
_BLK = 5000
_NBLK = _NUM_PIDS // _BLK


def _oim_kernel(lab_ref, inputs_ref, cls_ref, lut_ref, cq_ref, out_ref,
                m_ref, s_ref, picked_ref, safe_ref, valid_ref, x_ref):
    i = pl.program_id(0)

    @pl.when(i == 0)
    def _init():
        t_r = lab_ref[...] - 1
        row = jax.lax.broadcasted_iota(jnp.int32, (_B, _B), 0)
        col = jax.lax.broadcasted_iota(jnp.int32, (_B, _B), 1)
        diag = row == col
        t_mat = jnp.broadcast_to(t_r, (_B, _B))
        t_c = jnp.sum(jnp.where(diag, t_mat, 0), axis=0, keepdims=True)
        t_cmat = jnp.broadcast_to(t_c, (_B, _B))
        eq = t_mat == t_cmat
        earlier = col < row
        mask_r = t_r >= _NUM_PIDS
        any_earlier = jnp.sum((eq & earlier).astype(jnp.int32), axis=1,
                              keepdims=True) > 0
        first_r = mask_r & jnp.logical_not(any_earlier)
        first_c = jnp.sum(jnp.where(diag & jnp.broadcast_to(first_r, (_B, _B)),
                                    1, 0), axis=0, keepdims=True) > 0
        less = t_cmat < t_mat
        rank = jnp.sum((jnp.broadcast_to(first_c, (_B, _B)) & less)
                       .astype(jnp.int32), axis=1, keepdims=True)
        label = jnp.where(mask_r, _NUM_PIDS + rank % _NUM_CQ, t_r)
        valid = label != -1
        safe_ref[...] = jnp.where(valid, label, 0)
        valid_ref[...] = valid.astype(jnp.float32)
        m_ref[...] = jnp.full((_B, 1), -1e30, jnp.float32)
        s_ref[...] = jnp.zeros((_B, 1), jnp.float32)
        picked_ref[...] = jnp.zeros((_B, 1), jnp.float32)
        x_ref[...] = inputs_ref[...] * cls_ref[...]

    def _accumulate(logits, base):
        bm = jnp.max(logits, axis=1, keepdims=True)
        m_old = m_ref[...]
        m_new = jnp.maximum(m_old, bm)
        p = jnp.exp(logits - m_new)
        s_ref[...] = (s_ref[...] * jnp.exp(m_old - m_new)
                      + jnp.sum(p, axis=1, keepdims=True))
        m_ref[...] = m_new
        cols = jax.lax.broadcasted_iota(jnp.int32, logits.shape, 1) + base
        sel = cols == safe_ref[...]
        picked_ref[...] += jnp.sum(jnp.where(sel, logits, 0.0), axis=1,
                                   keepdims=True)

    x = x_ref[...]
    logits = jax.lax.dot_general(
        x, lut_ref[...], (((1,), (1,)), ((), ())),
        preferred_element_type=jnp.float32) * _OIM_SCALAR
    _accumulate(logits, i * _BLK)

    @pl.when(i == _NBLK - 1)
    def _final():
        cq_logits = jax.lax.dot_general(
            x, cq_ref[...], (((1,), (1,)), ((), ())),
            preferred_element_type=jnp.float32) * _OIM_SCALAR
        _accumulate(cq_logits, _NUM_PIDS)
        lse = m_ref[...] + jnp.log(s_ref[...])
        nll = lse - picked_ref[...]
        valid = valid_ref[...]
        cnt = jnp.sum(valid, axis=0, keepdims=True)
        total = jnp.sum(nll * valid, axis=0, keepdims=True)
        out_ref[...] = total / jnp.maximum(cnt, 1.0)


def kernel(inputs, roi_label, cls_scores, images, proposals, GT_info, lut, cq):
    del images, proposals, GT_info
    lab = roi_label.reshape(_B, 1).astype(jnp.int32)
    out = pl.pallas_call(
        _oim_kernel,
        grid=(_NBLK,),
        in_specs=[
            pl.BlockSpec((_B, 1), lambda i: (0, 0)),
            pl.BlockSpec((_B, _NUM_FEATURES), lambda i: (0, 0)),
            pl.BlockSpec((_B, 1), lambda i: (0, 0)),
            pl.BlockSpec((_BLK, _NUM_FEATURES), lambda i: (i, 0)),
            pl.BlockSpec((_NUM_CQ, _NUM_FEATURES), lambda i: (0, 0)),
        ],
        out_specs=pl.BlockSpec((1, 1), lambda i: (0, 0)),
        out_shape=jax.ShapeDtypeStruct((1, 1), jnp.float32),
        scratch_shapes=[
            pltpu.VMEM((_B, 1), jnp.float32),
            pltpu.VMEM((_B, 1), jnp.float32),
            pltpu.VMEM((_B, 1), jnp.float32),
            pltpu.VMEM((_B, 1), jnp.int32),
            pltpu.VMEM((_B, 1), jnp.float32),
            pltpu.VMEM((_B, _NUM_FEATURES), jnp.float32),
        ],
        compiler_params=pltpu.CompilerParams(
            dimension_semantics=("arbitrary",)),
    )(lab, inputs, cls_scores, lut, cq)
    return out[0, 0]


# speedup vs baseline: 1.6357x; 1.0194x over previous
import math

import jax
import jax.numpy as jnp
from jax.experimental import pallas as pl
from jax.experimental.pallas import tpu as pltpu

_NUM_FEATURES = 128
_NUM_PIDS = 100000
_NUM_CQ = 5000
_OIM_SCALAR = 30.0
_B = 128
_BLK = 5000
_NBLK = _NUM_PIDS // _BLK
_LOG2E = math.log2(math.e)
_LN2 = math.log(2.0)


def _oim_kernel(lab_ref, inputs_ref, cls_ref, lut_ref, cq_ref, out_ref,
                m_ref, s_ref, picked_ref, safe_ref, valid_ref, x_ref,
                iota_ref):
    i = pl.program_id(0)

    @pl.when(i == 0)
    def _init():
        t_r = lab_ref[...] - 1
        row = jax.lax.broadcasted_iota(jnp.int32, (_B, _B), 0)
        col = jax.lax.broadcasted_iota(jnp.int32, (_B, _B), 1)
        diag = row == col
        t_mat = jnp.broadcast_to(t_r, (_B, _B))
        t_c = jnp.sum(jnp.where(diag, t_mat, 0), axis=0, keepdims=True)
        t_cmat = jnp.broadcast_to(t_c, (_B, _B))
        eq = t_mat == t_cmat
        earlier = col < row
        mask_r = t_r >= _NUM_PIDS
        any_earlier = jnp.sum((eq & earlier).astype(jnp.int32), axis=1,
                              keepdims=True) > 0
        first_r = mask_r & jnp.logical_not(any_earlier)
        first_c = jnp.sum(jnp.where(diag & jnp.broadcast_to(first_r, (_B, _B)),
                                    1, 0), axis=0, keepdims=True) > 0
        less = t_cmat < t_mat
        rank = jnp.sum((jnp.broadcast_to(first_c, (_B, _B)) & less)
                       .astype(jnp.int32), axis=1, keepdims=True)
        label = jnp.where(mask_r, _NUM_PIDS + rank % _NUM_CQ, t_r)
        valid = label != -1
        safe_ref[...] = jnp.where(valid, label, 0)
        valid_ref[...] = valid.astype(jnp.float32)
        m_ref[...] = jnp.full((_B, 1), -1e30, jnp.float32)
        s_ref[...] = jnp.zeros((_B, 1), jnp.float32)
        picked_ref[...] = jnp.zeros((_B, 1), jnp.float32)
        x_ref[...] = inputs_ref[...] * (cls_ref[...] * (_OIM_SCALAR * _LOG2E))
        iota_ref[...] = jax.lax.broadcasted_iota(jnp.int32, (_B, _BLK), 1)

    def _accumulate(logits, base):
        bm = jnp.max(logits, axis=1, keepdims=True)
        m_old = m_ref[...]
        m_new = jnp.maximum(m_old, bm)
        p = jnp.exp2(logits - m_new)
        s_ref[...] = (s_ref[...] * jnp.exp2(m_old - m_new)
                      + jnp.sum(p, axis=1, keepdims=True))
        m_ref[...] = m_new
        sel = iota_ref[...] == safe_ref[...] - base
        picked_ref[...] += jnp.sum(jnp.where(sel, logits, 0.0), axis=1,
                                   keepdims=True)

    x = x_ref[...]
    logits = jax.lax.dot_general(
        x, lut_ref[...], (((1,), (1,)), ((), ())),
        preferred_element_type=jnp.float32)
    _accumulate(logits, i * _BLK)

    @pl.when(i == _NBLK - 1)
    def _final():
        cq_logits = jax.lax.dot_general(
            x, cq_ref[...], (((1,), (1,)), ((), ())),
            preferred_element_type=jnp.float32)
        _accumulate(cq_logits, _NUM_PIDS)
        lse2 = m_ref[...] + jnp.log2(s_ref[...])
        nll = (lse2 - picked_ref[...]) * _LN2
        valid = valid_ref[...]
        cnt = jnp.sum(valid, axis=0, keepdims=True)
        total = jnp.sum(nll * valid, axis=0, keepdims=True)
        out_ref[...] = total / jnp.maximum(cnt, 1.0)


def kernel(inputs, roi_label, cls_scores, images, proposals, GT_info, lut, cq):
    del images, proposals, GT_info
    lab = roi_label.reshape(_B, 1).astype(jnp.int32)
    out = pl.pallas_call(
        _oim_kernel,
        grid=(_NBLK,),
        in_specs=[
            pl.BlockSpec((_B, 1), lambda i: (0, 0)),
            pl.BlockSpec((_B, _NUM_FEATURES), lambda i: (0, 0)),
            pl.BlockSpec((_B, 1), lambda i: (0, 0)),
            pl.BlockSpec((_BLK, _NUM_FEATURES), lambda i: (i, 0)),
            pl.BlockSpec((_NUM_CQ, _NUM_FEATURES), lambda i: (0, 0)),
        ],
        out_specs=pl.BlockSpec((1, 1), lambda i: (0, 0)),
        out_shape=jax.ShapeDtypeStruct((1, 1), jnp.float32),
        scratch_shapes=[
            pltpu.VMEM((_B, 1), jnp.float32),
            pltpu.VMEM((_B, 1), jnp.float32),
            pltpu.VMEM((_B, 1), jnp.float32),
            pltpu.VMEM((_B, 1), jnp.int32),
            pltpu.VMEM((_B, 1), jnp.float32),
            pltpu.VMEM((_B, _NUM_FEATURES), jnp.float32),
            pltpu.VMEM((_B, _BLK), jnp.int32),
        ],
        compiler_params=pltpu.CompilerParams(
            dimension_semantics=("arbitrary",)),
    )(lab, inputs, cls_scores, lut, cq)
    return out[0, 0]
